# baseline (speedup 1.0000x reference)
import functools

import jax
import jax.numpy as jnp
from jax import lax
from jax.experimental import pallas as pl
from jax.experimental.pallas import tpu as pltpu
from jax.experimental.pallas import tpu_sc as plsc

L = 16
NS = 16
NUM_EXPERTS = 64


def _make_hist_kernel(n_rows: int, n_cols: int, num_experts: int):
  E = num_experts
  W = n_cols
  rows = n_rows // NS
  assert rows * NS == n_rows and W % L == 0 and E % L == 0

  mesh = plsc.VectorSubcoreMesh(
      core_axis_name="c", subcore_axis_name="s", num_cores=1, num_subcores=NS)

  @functools.partial(
      pl.kernel,
      out_type=jax.ShapeDtypeStruct((E,), jnp.int32),
      mesh=mesh,
      compiler_params=pltpu.CompilerParams(
          needs_layout_passes=False, use_tc_tiling_on_sc=False,
          skip_device_barrier=True),
      scratch_types=[
          pltpu.VMEM((rows, W), jnp.int32),
          pltpu.VMEM((2 * E * L + L,), jnp.int32),
          pltpu.VMEM((E,), jnp.int32),
          pltpu.VMEM((NS * E,), jnp.int32),
          pltpu.VMEM_SHARED((NS * E,), jnp.int32),
      ],
  )
  def hist_kernel(ids_hbm, out_hbm, ids_v, hist_v, cnt_v, gbuf_v, shared):
    sid = lax.axis_index("s")
    pltpu.sync_copy(ids_hbm.at[pl.ds(sid * rows, rows)], ids_v)

    lanes = lax.iota(jnp.int32, L)
    zeros = jnp.zeros((L,), jnp.int32)
    ones = jnp.ones((L,), jnp.int32)
    B2 = E * L + L
    for r in range(2 * E + 1):
      hist_v[pl.ds(r * L, L)] = zeros

    def body(r, carry):
      for c in range(W // L):
        v = ids_v[r, pl.ds(c * L, L)]
        off = (c % 2) * B2
        plsc.addupdate_scatter(hist_v, [v * L + (lanes + off)], ones)
      return carry

    lax.fori_loop(0, rows, body, 0)

    for k in range(E // L):
      acc = zeros
      for j in range(L):
        b = (k * L + j) * L
        s = jnp.sum(hist_v[pl.ds(b, L)] + hist_v[pl.ds(B2 + b, L)])
        acc = jnp.where(lanes == j, s, acc)
      cnt_v[pl.ds(k * L, L)] = acc

    pltpu.sync_copy(cnt_v, shared.at[pl.ds(sid * E, E)])
    plsc.subcore_barrier()

    @pl.when(sid == 0)
    def _():
      pltpu.sync_copy(shared, gbuf_v)
      for k in range(E // L):
        acc = zeros
        for s_ in range(NS):
          acc = acc + gbuf_v[pl.ds(s_ * E + k * L, L)]
        cnt_v[pl.ds(k * L, L)] = acc
      pltpu.sync_copy(cnt_v, out_hbm)

  return hist_kernel


def kernel(topk_ids, num_local_experts):
  del num_local_experts
  n, k = topk_ids.shape
  ids_t = topk_ids.reshape(n // 128, 128, k).transpose(0, 2, 1)
  ids_t = ids_t.reshape(n // 128 * k, 128)
  hist = _make_hist_kernel(ids_t.shape[0], ids_t.shape[1], NUM_EXPERTS)
  return hist(ids_t)


# speedup vs baseline: 1.0429x; 1.0429x over previous
import functools

import jax
import jax.numpy as jnp
from jax import lax
from jax.experimental import pallas as pl
from jax.experimental.pallas import tpu as pltpu
from jax.experimental.pallas import tpu_sc as plsc

L = 16
NS = 16
NUM_EXPERTS = 64


def _make_hist_kernel(n_rows: int, n_cols: int, num_experts: int):
  E = num_experts
  W = n_cols
  rows = n_rows // NS
  assert rows * NS == n_rows and W % L == 0 and E % L == 0

  mesh = plsc.VectorSubcoreMesh(
      core_axis_name="c", subcore_axis_name="s", num_cores=1, num_subcores=NS)

  @functools.partial(
      pl.kernel,
      out_type=jax.ShapeDtypeStruct((E,), jnp.int32),
      mesh=mesh,
      compiler_params=pltpu.CompilerParams(
          needs_layout_passes=False, use_tc_tiling_on_sc=False,
          skip_device_barrier=True),
      scratch_types=[
          pltpu.VMEM((rows, W), jnp.int32),
          pltpu.VMEM((E * L,), jnp.int32),
          pltpu.VMEM((E,), jnp.int32),
          pltpu.VMEM((NS * E,), jnp.int32),
          pltpu.VMEM_SHARED((NS * E,), jnp.int32),
      ],
  )
  def hist_kernel(ids_hbm, out_hbm, ids_v, hist_v, cnt_v, gbuf_v, shared):
    sid = lax.axis_index("s")
    pltpu.sync_copy(ids_hbm.at[pl.ds(sid * rows, rows)], ids_v)

    lanes = lax.iota(jnp.int32, L)
    zeros = jnp.zeros((L,), jnp.int32)
    ones = jnp.ones((L,), jnp.int32)
    for r in range(E):
      hist_v[pl.ds(r * L, L)] = zeros

    def body(r, carry):
      for c in range(W // L):
        v = ids_v[r, pl.ds(c * L, L)]
        plsc.addupdate_scatter(hist_v, [v * L + lanes], ones)
      return carry

    lax.fori_loop(0, rows, body, 0)

    for k in range(E // L):
      acc = zeros
      for j in range(L):
        s = jnp.sum(hist_v[pl.ds((k * L + j) * L, L)])
        acc = jnp.where(lanes == j, s, acc)
      cnt_v[pl.ds(k * L, L)] = acc

    pltpu.sync_copy(cnt_v, shared.at[pl.ds(sid * E, E)])
    plsc.subcore_barrier()

    @pl.when(sid == 0)
    def _():
      pltpu.sync_copy(shared, gbuf_v)
      for k in range(E // L):
        acc = zeros
        for s_ in range(NS):
          acc = acc + gbuf_v[pl.ds(s_ * E + k * L, L)]
        cnt_v[pl.ds(k * L, L)] = acc
      pltpu.sync_copy(cnt_v, out_hbm)

  return hist_kernel


def kernel(topk_ids, num_local_experts):
  del num_local_experts
  n, k = topk_ids.shape
  ids_t = topk_ids.reshape(n // 128, 128, k).transpose(0, 2, 1)
  ids_t = ids_t.reshape(n // 128 * k, 128)
  hist = _make_hist_kernel(ids_t.shape[0], ids_t.shape[1], NUM_EXPERTS)
  return hist(ids_t)


# speedup vs baseline: 1.0442x; 1.0012x over previous
import functools

import jax
import jax.numpy as jnp
from jax import lax
from jax.experimental import pallas as pl
from jax.experimental.pallas import tpu as pltpu
from jax.experimental.pallas import tpu_sc as plsc

L = 16
NS = 16
NUM_EXPERTS = 64


def _make_hist_kernel(n_rows: int, n_cols: int, num_experts: int):
  E = num_experts
  W = n_cols
  rows = n_rows // NS
  assert rows * NS == n_rows and W % L == 0 and E % L == 0

  mesh = plsc.VectorSubcoreMesh(
      core_axis_name="c", subcore_axis_name="s", num_cores=1, num_subcores=NS)

  @functools.partial(
      pl.kernel,
      out_type=jax.ShapeDtypeStruct((E,), jnp.int32),
      mesh=mesh,
      compiler_params=pltpu.CompilerParams(
          needs_layout_passes=False, use_tc_tiling_on_sc=False,
          skip_device_barrier=True),
      scratch_types=[
          pltpu.VMEM((rows, W), jnp.int32),
          pltpu.VMEM((E * L,), jnp.int32),
          pltpu.VMEM((E,), jnp.int32),
          pltpu.VMEM((NS * E,), jnp.int32),
          pltpu.VMEM_SHARED((NS * E,), jnp.int32),
          pltpu.SemaphoreType.DMA,
          pltpu.SemaphoreType.DMA,
      ],
  )
  def hist_kernel(ids_hbm, out_hbm, ids_v, hist_v, cnt_v, gbuf_v, shared,
                  sem0, sem1):
    sid = lax.axis_index("s")
    half = rows // 2
    base = sid * rows
    cp0 = pltpu.make_async_copy(
        ids_hbm.at[pl.ds(base, half)], ids_v.at[pl.ds(0, half)], sem0)
    cp0.start()
    cp1 = pltpu.make_async_copy(
        ids_hbm.at[pl.ds(base + half, half)], ids_v.at[pl.ds(half, half)],
        sem1)
    cp1.start()

    lanes = lax.iota(jnp.int32, L)
    zeros = jnp.zeros((L,), jnp.int32)
    ones = jnp.ones((L,), jnp.int32)
    for r in range(E):
      hist_v[pl.ds(r * L, L)] = zeros

    def body(r, carry):
      for u in range(2):
        for c in range(W // L):
          v = ids_v[2 * r + u, pl.ds(c * L, L)]
          plsc.addupdate_scatter(hist_v, [v * L + lanes], ones)
      return carry

    cp0.wait()
    lax.fori_loop(0, half // 2, body, 0)
    cp1.wait()
    lax.fori_loop(half // 2, rows // 2, body, 0)

    for k in range(E // L):
      acc = zeros
      for j in range(L):
        s = jnp.sum(hist_v[pl.ds((k * L + j) * L, L)])
        acc = jnp.where(lanes == j, s, acc)
      cnt_v[pl.ds(k * L, L)] = acc

    pltpu.sync_copy(cnt_v, shared.at[pl.ds(sid * E, E)])
    plsc.subcore_barrier()

    @pl.when(sid == 0)
    def _():
      pltpu.sync_copy(shared, gbuf_v)
      for k in range(E // L):
        acc = zeros
        for s_ in range(NS):
          acc = acc + gbuf_v[pl.ds(s_ * E + k * L, L)]
        cnt_v[pl.ds(k * L, L)] = acc
      pltpu.sync_copy(cnt_v, out_hbm)

  return hist_kernel


def kernel(topk_ids, num_local_experts):
  del num_local_experts
  n, k = topk_ids.shape
  ids_t = topk_ids.reshape(n // 128, 128, k).transpose(0, 2, 1)
  ids_t = ids_t.reshape(n // 128 * k, 128)
  hist = _make_hist_kernel(ids_t.shape[0], ids_t.shape[1], NUM_EXPERTS)
  return hist(ids_t)


# speedup vs baseline: 1.0449x; 1.0008x over previous
import functools

import jax
import jax.numpy as jnp
from jax import lax
from jax.experimental import pallas as pl
from jax.experimental.pallas import tpu as pltpu
from jax.experimental.pallas import tpu_sc as plsc

L = 16
NS = 16
NUM_EXPERTS = 64


def _make_hist_kernel(n_rows: int, n_cols: int, num_experts: int):
  E = num_experts
  W = n_cols
  rows = n_rows // NS
  assert rows * NS == n_rows and W % L == 0 and E % L == 0

  mesh = plsc.VectorSubcoreMesh(
      core_axis_name="c", subcore_axis_name="s", num_cores=1, num_subcores=NS)

  @functools.partial(
      pl.kernel,
      out_type=jax.ShapeDtypeStruct((E,), jnp.int32),
      mesh=mesh,
      compiler_params=pltpu.CompilerParams(
          needs_layout_passes=False, use_tc_tiling_on_sc=False,
          skip_device_barrier=True),
      scratch_types=[
          pltpu.VMEM((rows, W), jnp.int32),
          pltpu.VMEM((E * L,), jnp.int32),
          pltpu.VMEM((E,), jnp.int32),
          pltpu.VMEM((NS * E,), jnp.int32),
          pltpu.VMEM_SHARED((NS * E,), jnp.int32),
      ],
  )
  def hist_kernel(ids_hbm, out_hbm, ids_v, hist_v, cnt_v, gbuf_v, shared):
    sid = lax.axis_index("s")
    pltpu.sync_copy(ids_hbm.at[pl.ds(sid * rows, rows)], ids_v)

    lanes = lax.iota(jnp.int32, L)
    zeros = jnp.zeros((L,), jnp.int32)
    ones = jnp.ones((L,), jnp.int32)
    for r in range(E):
      hist_v[pl.ds(r * L, L)] = zeros

    def body(r, carry):
      for c in range(W // L):
        v = ids_v[r, pl.ds(c * L, L)]
        plsc.addupdate_scatter(hist_v, [v * L + lanes], ones)
      return carry

    lax.fori_loop(0, rows, body, 0)

    for k in range(E // L):
      acc = zeros
      for j in range(L):
        s = jnp.sum(hist_v[pl.ds((k * L + j) * L, L)])
        acc = jnp.where(lanes == j, s, acc)
      cnt_v[pl.ds(k * L, L)] = acc

    pltpu.sync_copy(cnt_v, shared.at[pl.ds(sid * E, E)])
    plsc.subcore_barrier()

    @pl.when(sid == 0)
    def _():
      pltpu.sync_copy(shared, gbuf_v)
      for k in range(E // L):
        acc = zeros
        for s_ in range(NS):
          acc = acc + gbuf_v[pl.ds(s_ * E + k * L, L)]
        cnt_v[pl.ds(k * L, L)] = acc
      pltpu.sync_copy(cnt_v, out_hbm)

  return hist_kernel


def kernel(topk_ids, num_local_experts):
  del num_local_experts
  n, k = topk_ids.shape
  ids_t = topk_ids.reshape(n // 128, 128, k).transpose(0, 2, 1)
  ids_t = ids_t.reshape(n // 128 * k, 128)
  hist = _make_hist_kernel(ids_t.shape[0], ids_t.shape[1], NUM_EXPERTS)
  return hist(ids_t)
